# baseline (speedup 1.0000x reference)
import functools

import jax
import jax.numpy as jnp
from jax import lax
from jax.experimental import pallas as pl
from jax.experimental.pallas import tpu as pltpu
from jax.experimental.pallas import tpu_sc as plsc


def _sc_gather(src, idx, n_rows, n_cols):
    info = plsc.get_sparse_core_info()
    nw = info.num_cores * info.num_subcores
    rows_pw = 8
    while n_rows % (rows_pw * 2) == 0 and (n_rows // (rows_pw * 2)) > nw // 2:
        rows_pw *= 2
    while n_rows % rows_pw != 0:
        rows_pw += 8
    active = n_rows // rows_pw
    nc = info.num_cores
    mesh = plsc.VectorSubcoreMesh(core_axis_name="c", subcore_axis_name="s")

    @functools.partial(
        pl.kernel,
        out_type=jax.ShapeDtypeStruct((n_rows, n_cols), jnp.float32),
        mesh=mesh,
        scratch_types=[
            pltpu.VMEM((rows_pw,), jnp.int32),
            pltpu.VMEM((rows_pw, n_cols), jnp.float32),
            pltpu.SemaphoreType.DMA,
        ],
    )
    def gather_kernel(src_hbm, idx_hbm, out_hbm, idx_v, rows_v, sem):
        wid = lax.axis_index("s") * nc + lax.axis_index("c")

        @pl.when(wid < active)
        def _():
            base = wid * rows_pw
            pltpu.sync_copy(idx_hbm.at[pl.ds(base, rows_pw)], idx_v)
            pltpu.async_copy(src_hbm.at[idx_v], rows_v, sem).wait()
            pltpu.sync_copy(rows_v, out_hbm.at[pl.ds(base, rows_pw)])

    return gather_kernel(src, idx)


def _sc_add(tokens, table):
    b, h, w, t, bs, d = tokens.shape
    info = plsc.get_sparse_core_info()
    nc = info.num_cores
    nw = nc * info.num_subcores
    th = t // 2
    n_combo = b * 2
    grp_per_combo = nw // n_combo
    sites_pg = (h * w) // grp_per_combo
    mesh = plsc.VectorSubcoreMesh(core_axis_name="c", subcore_axis_name="s")
    cshape = (th, bs, d)

    nbuf = 4

    @functools.partial(
        pl.kernel,
        out_type=jax.ShapeDtypeStruct(tokens.shape, tokens.dtype),
        mesh=mesh,
        scratch_types=[pltpu.VMEM(cshape, jnp.float32)] * (1 + nbuf)
        + [pltpu.SemaphoreType.DMA] * (2 * nbuf),
    )
    def add_kernel(tok_hbm, tab_hbm, out_hbm, tab_v, *bufsem):
        bufs = bufsem[:nbuf]
        sin = bufsem[nbuf:2 * nbuf]
        sout = bufsem[2 * nbuf:]
        wid = lax.axis_index("s") * nc + lax.axis_index("c")
        combo = wid // grp_per_combo
        grp = wid % grp_per_combo
        bi = combo // 2
        t0 = (combo % 2) * th
        base = grp * sites_pg
        pltpu.sync_copy(tab_hbm.at[bi, pl.ds(t0, th)], tab_v)

        def chunk_at(ref, s):
            return ref.at[bi, s // w, s % w, pl.ds(t0, th)]

        for k in range(nbuf - 1):
            pltpu.async_copy(chunk_at(tok_hbm, base + k), bufs[k], sin[k])

        @pl.loop(0, sites_pg, step=nbuf)
        def _sites(s0):
            for k in range(nbuf):
                s = s0 + k
                buf, s_in, s_out = bufs[k], sin[k], sout[k]
                fk = (k + nbuf - 1) % nbuf

                @pl.when(s + nbuf - 1 < sites_pg)
                def _():
                    @pl.when(s >= 1)
                    def _():
                        pltpu.make_async_copy(
                            bufs[fk], chunk_at(out_hbm, base), sout[fk]
                        ).wait()

                    pltpu.async_copy(
                        chunk_at(tok_hbm, base + s + nbuf - 1), bufs[fk], sin[fk]
                    )

                pltpu.make_async_copy(chunk_at(tok_hbm, base), buf, s_in).wait()

                @pl.loop(0, d // 16, unroll=4)
                def _add(j):
                    sl = pl.ds(pl.multiple_of(j * 16, 16), 16)
                    for r in range(th):
                        for c in range(bs):
                            buf[r, c, sl] = buf[r, c, sl] + tab_v[r, c, sl]

                pltpu.async_copy(buf, chunk_at(out_hbm, base + s), s_out)

        for k in range(nbuf):
            pltpu.make_async_copy(
                bufs[k], chunk_at(out_hbm, base), sout[k]
            ).wait()

    return add_kernel(tokens, table)


def kernel(tokens, timestamps, channel_embed, pos_embed, month_table):
    b, h, w, t, bs, d = tokens.shape
    n = d // 4
    months = timestamps[:, :, 1]

    off_pos = channel_embed.shape[0]
    off_mon = off_pos + pos_embed.shape[0]
    off_zero = off_mon + month_table.shape[0]
    src = jnp.concatenate(
        [channel_embed, pos_embed, month_table, jnp.zeros((1, n), jnp.float32)],
        axis=0,
    )
    n_pad = ((n + 127) // 128) * 128
    src = jnp.pad(src, ((0, 0), (0, n_pad - n)))

    i32 = jnp.int32
    q0 = jnp.broadcast_to(jnp.arange(bs, dtype=i32)[None, None, :], (b, t, bs))
    q1 = jnp.broadcast_to(
        off_pos + jnp.arange(t, dtype=i32)[None, :, None], (b, t, bs)
    )
    q2 = jnp.broadcast_to((off_mon + months.astype(i32))[:, :, None], (b, t, bs))
    q3 = jnp.full((b, t, bs), off_zero, dtype=i32)
    idx = jnp.stack([q0, q1, q2, q3], axis=-1).reshape(-1)

    table = _sc_gather(src, idx, b * t * bs * 4, n_pad)
    table = table[:, :n].reshape(b, t, bs, d)

    return _sc_add(tokens, table)


# speedup vs baseline: 2.0070x; 2.0070x over previous
import functools

import jax
import jax.numpy as jnp
from jax import lax
from jax.experimental import pallas as pl
from jax.experimental.pallas import tpu as pltpu
from jax.experimental.pallas import tpu_sc as plsc


def _sc_gather(src, idx, n_rows, n_cols):
    info = plsc.get_sparse_core_info()
    nw = info.num_cores * info.num_subcores
    rows_pw = 8
    while n_rows % (rows_pw * 2) == 0 and (n_rows // (rows_pw * 2)) > nw // 2:
        rows_pw *= 2
    while n_rows % rows_pw != 0:
        rows_pw += 8
    active = n_rows // rows_pw
    nc = info.num_cores
    mesh = plsc.VectorSubcoreMesh(core_axis_name="c", subcore_axis_name="s")

    @functools.partial(
        pl.kernel,
        out_type=jax.ShapeDtypeStruct((n_rows, n_cols), jnp.float32),
        mesh=mesh,
        scratch_types=[
            pltpu.VMEM((rows_pw,), jnp.int32),
            pltpu.VMEM((rows_pw, n_cols), jnp.float32),
            pltpu.SemaphoreType.DMA,
        ],
    )
    def gather_kernel(src_hbm, idx_hbm, out_hbm, idx_v, rows_v, sem):
        wid = lax.axis_index("s") * nc + lax.axis_index("c")

        @pl.when(wid < active)
        def _():
            base = wid * rows_pw
            pltpu.sync_copy(idx_hbm.at[pl.ds(base, rows_pw)], idx_v)
            pltpu.async_copy(src_hbm.at[idx_v], rows_v, sem).wait()
            pltpu.sync_copy(rows_v, out_hbm.at[pl.ds(base, rows_pw)])

    return gather_kernel(src, idx)


def _add_body(tok_ref, tab_ref, out_ref):
    out_ref[...] = tok_ref[...] + tab_ref[...][:, None]


def _tc_broadcast_add(tokens, table):
    b, hw, t, bs, d = tokens.shape
    ch = 64
    grid = (b, hw // ch)
    return pl.pallas_call(
        _add_body,
        out_shape=jax.ShapeDtypeStruct(tokens.shape, tokens.dtype),
        grid=grid,
        in_specs=[
            pl.BlockSpec((1, ch, t, bs, d), lambda i, j: (i, j, 0, 0, 0)),
            pl.BlockSpec((1, t, bs, d), lambda i, j: (i, 0, 0, 0)),
        ],
        out_specs=pl.BlockSpec((1, ch, t, bs, d), lambda i, j: (i, j, 0, 0, 0)),
        compiler_params=pltpu.CompilerParams(
            dimension_semantics=("parallel", "parallel")
        ),
    )(tokens, table)


def kernel(tokens, timestamps, channel_embed, pos_embed, month_table):
    b, h, w, t, bs, d = tokens.shape
    n = d // 4
    months = timestamps[:, :, 1]

    off_pos = channel_embed.shape[0]
    off_mon = off_pos + pos_embed.shape[0]
    off_zero = off_mon + month_table.shape[0]
    src = jnp.concatenate(
        [channel_embed, pos_embed, month_table, jnp.zeros((1, n), jnp.float32)],
        axis=0,
    )
    n_pad = ((n + 127) // 128) * 128
    src = jnp.pad(src, ((0, 0), (0, n_pad - n)))

    i32 = jnp.int32
    q0 = jnp.broadcast_to(jnp.arange(bs, dtype=i32)[None, None, :], (b, t, bs))
    q1 = jnp.broadcast_to(
        off_pos + jnp.arange(t, dtype=i32)[None, :, None], (b, t, bs)
    )
    q2 = jnp.broadcast_to((off_mon + months.astype(i32))[:, :, None], (b, t, bs))
    q3 = jnp.full((b, t, bs), off_zero, dtype=i32)
    idx = jnp.stack([q0, q1, q2, q3], axis=-1).reshape(-1)

    table = _sc_gather(src, idx, b * t * bs * 4, n_pad)
    table = table[:, :n].reshape(b, t, bs, d)

    tok = tokens.reshape(b, h * w, t, bs, d)
    out = _tc_broadcast_add(tok, table)
    return out.reshape(tokens.shape)
